# Initial kernel scaffold; baseline (speedup 1.0000x reference)
import jax
import jax.numpy as jnp
from jax.experimental import pallas as pl


def kernel(x_malware, x_domain, ei_delivers, ei_rev_delivers, ei_metapath_sim, params):
    raise NotImplementedError("write your pallas kernel here")


# speedup vs baseline: 1.0627x; 1.0627x over previous
import functools
import jax
import jax.numpy as jnp
from jax.experimental import pallas as pl
from jax.experimental.pallas import tpu as pltpu

H = 128
HEADS = 4
CH = 32
BS = 2000


def _row(bs, w):
    return pl.BlockSpec((bs, w), lambda i: (i, 0))


def _rep(h, w):
    return pl.BlockSpec((h, w), lambda i: (0, 0))


def _ln(x, g, b):
    m = jnp.mean(x, axis=-1, keepdims=True)
    v = jnp.mean((x - m) ** 2, axis=-1, keepdims=True)
    return (x - m) / jnp.sqrt(v + 1e-5) * g + b


def _proj_body(xm, Wm, bm, xd, Wd, bd, hm, hd):
    hm[...] = jnp.dot(xm[...], Wm[...], preferred_element_type=jnp.float32) + bm[...]
    hd[...] = jnp.dot(xd[...], Wd[...], preferred_element_type=jnp.float32) + bd[...]


def _proj(xm, Wm, bm, xd, Wd, bd):
    n = xm.shape[0]
    return pl.pallas_call(
        _proj_body,
        grid=(n // BS,),
        in_specs=[_row(BS, H), _rep(H, H), _rep(1, H),
                  _row(BS, H), _rep(H, H), _rep(1, H)],
        out_specs=[_row(BS, H), _row(BS, H)],
        out_shape=[jax.ShapeDtypeStruct((n, H), jnp.float32)] * 2,
    )(xm, Wm, bm.reshape(1, H), xd, Wd, bd.reshape(1, H))


def _sage_post(o):
    nrm = jnp.sqrt(jnp.sum(o * o, axis=-1, keepdims=True))
    return jax.nn.relu(o / jnp.maximum(nrm, 1e-12))


def _dom_body(res, agg, invc, hdom, Wl, bl, Wr, g, b, rdom, out):
    mean = agg[...] * invc[...]
    o = (jnp.dot(mean, Wl[...], preferred_element_type=jnp.float32) + bl[...]
         + jnp.dot(hdom[...], Wr[...], preferred_element_type=jnp.float32))
    h = _ln(_sage_post(o), g[...], b[...])
    if res:
        h = h + rdom[...]
    out[...] = h


def _dom_stage(agg, invc, hdom, Wl, bl, Wr, g, b, rdom, res):
    n = agg.shape[0]
    return pl.pallas_call(
        functools.partial(_dom_body, res),
        grid=(n // BS,),
        in_specs=[_row(BS, H), _row(BS, 1), _row(BS, H), _rep(H, H),
                  _rep(1, H), _rep(H, H), _rep(1, H), _rep(1, H), _row(BS, H)],
        out_specs=_row(BS, H),
        out_shape=jax.ShapeDtypeStruct((n, H), jnp.float32),
    )(agg, invc, hdom, Wl, bl.reshape(1, H), Wr, g.reshape(1, H),
      b.reshape(1, H), rdom)


def _gat_pre_body(hmal, W, A8, xl, asd):
    v = jnp.dot(hmal[...], W[...], preferred_element_type=jnp.float32)
    xl[...] = v
    asd[...] = jnp.dot(v, A8[...], preferred_element_type=jnp.float32)


def _gat_pre(hmal, W, A8):
    n = hmal.shape[0]
    return pl.pallas_call(
        _gat_pre_body,
        grid=(n // BS,),
        in_specs=[_row(BS, H), _rep(H, H), _rep(H, 2 * HEADS)],
        out_specs=[_row(BS, H), _row(BS, 2 * HEADS)],
        out_shape=[jax.ShapeDtypeStruct((n, H), jnp.float32),
                   jax.ShapeDtypeStruct((n, 2 * HEADS), jnp.float32)],
    )(hmal, W, A8)


def _mal_body(res, aggr, invc, hmal, gacc, invden, S, Wl, bl, Wr, bmeta,
              W1a, W1b, b1, W2, b2, g, b, rmal, out):
    mean = aggr[...] * invc[...]
    o = (jnp.dot(mean, Wl[...], preferred_element_type=jnp.float32) + bl[...]
         + jnp.dot(hmal[...], Wr[...], preferred_element_type=jnp.float32))
    o_mal = _sage_post(o)
    den128 = jnp.dot(invden[...], S[...], preferred_element_type=jnp.float32)
    m_mal = jax.nn.relu(gacc[...] * den128 + bmeta[...])
    hmid = jax.nn.relu(
        jnp.dot(o_mal, W1a[...], preferred_element_type=jnp.float32)
        + jnp.dot(m_mal, W1b[...], preferred_element_type=jnp.float32)
        + b1[...])
    f = jnp.dot(hmid, W2[...], preferred_element_type=jnp.float32) + b2[...]
    h = _ln(f, g[...], b[...])
    if res:
        h = h + rmal[...]
    out[...] = h


def _mal_stage(aggr, invc, hmal, gacc, invden, S, Wl, bl, Wr, bmeta,
               W1, b1, W2, b2, g, b, rmal, res):
    n = aggr.shape[0]
    W1a, W1b = W1[:H], W1[H:]
    return pl.pallas_call(
        functools.partial(_mal_body, res),
        grid=(n // BS,),
        in_specs=[_row(BS, H), _row(BS, 1), _row(BS, H), _row(BS, H),
                  _row(BS, HEADS), _rep(HEADS, H), _rep(H, H), _rep(1, H),
                  _rep(H, H), _rep(1, H), _rep(H, H), _rep(H, H), _rep(1, H),
                  _rep(H, H), _rep(1, H), _rep(1, H), _rep(1, H), _row(BS, H)],
        out_specs=_row(BS, H),
        out_shape=jax.ShapeDtypeStruct((n, H), jnp.float32),
    )(aggr, invc, hmal, gacc, invden, S, Wl, bl.reshape(1, H), Wr,
      bmeta.reshape(1, H), W1a, W1b, b1.reshape(1, H), W2, b2.reshape(1, H),
      g.reshape(1, H), b.reshape(1, H), rmal)


def _seg_sum_rows(vals, idx, n):
    return jax.ops.segment_sum(vals, idx, num_segments=n)


def _inv_counts(idx, n):
    c = jax.ops.segment_sum(jnp.ones(idx.shape, jnp.float32), idx,
                            num_segments=n)
    return (1.0 / jnp.maximum(c, 1.0)).reshape(n, 1)


def kernel(x_malware, x_domain, ei_delivers, ei_rev_delivers,
           ei_metapath_sim, params):
    p = params
    nm, nd = x_malware.shape[0], x_domain.shape[0]

    src_del, dst_del = ei_delivers[0], ei_delivers[1]
    src_rev, dst_rev = ei_rev_delivers[0], ei_rev_delivers[1]
    src_meta, dst_meta = ei_metapath_sim[0], ei_metapath_sim[1]

    invc_del = _inv_counts(dst_del, nd)
    invc_rev = _inv_counts(dst_rev, nm)

    eye = jnp.eye(HEADS, dtype=jnp.float32)
    As = jnp.einsum('hc,hk->hck', p['meta0_as'], eye).reshape(H, HEADS)
    Ad = jnp.einsum('hc,hk->hck', p['meta0_ad'], eye).reshape(H, HEADS)
    A8_0 = jnp.concatenate([As, Ad], axis=1)
    As1 = jnp.einsum('hc,hk->hck', p['meta1_as'], eye).reshape(H, HEADS)
    Ad1 = jnp.einsum('hc,hk->hck', p['meta1_ad'], eye).reshape(H, HEADS)
    A8_1 = jnp.concatenate([As1, Ad1], axis=1)
    A8s = [A8_0, A8_1]
    S = jnp.kron(eye, jnp.ones((1, CH), jnp.float32))

    h_mal, h_dom = _proj(x_malware, p['proj_mal_W'], p['proj_mal_b'],
                         x_domain, p['proj_dom_W'], p['proj_dom_b'])
    r_mal, r_dom = h_mal, h_dom

    for i in range(2):
        agg_del = _seg_sum_rows(h_mal[src_del], dst_del, nd)
        agg_rev = _seg_sum_rows(h_dom[src_rev], dst_rev, nm)

        xl, asd = _gat_pre(h_mal, p['meta%d_W' % i], A8s[i])
        a_s, a_d = asd[:, :HEADS], asd[:, HEADS:]
        alpha = jax.nn.leaky_relu(a_s[src_meta] + a_d[dst_meta], 0.2)
        ex = jnp.exp(alpha)
        den = _seg_sum_rows(ex, dst_meta, nm)
        invden = 1.0 / (den + 1e-16)
        xl4 = xl.reshape(nm, HEADS, CH)
        gacc = _seg_sum_rows(xl4[src_meta] * ex[:, :, None], dst_meta,
                             nm).reshape(nm, H)

        new_dom = _dom_stage(agg_del, invc_del, h_dom, p['del%d_Wl' % i],
                             p['del%d_bl' % i], p['del%d_Wr' % i],
                             p['ln_dom%d_g' % i], p['ln_dom%d_b' % i],
                             r_dom, i > 0)
        new_mal = _mal_stage(agg_rev, invc_rev, h_mal, gacc, invden, S,
                             p['rev%d_Wl' % i], p['rev%d_bl' % i],
                             p['rev%d_Wr' % i], p['meta%d_b' % i],
                             p['fus%d_W1' % i], p['fus%d_b1' % i],
                             p['fus%d_W2' % i], p['fus%d_b2' % i],
                             p['ln_mal%d_g' % i], p['ln_mal%d_b' % i],
                             r_mal, i > 0)
        h_mal, h_dom = new_mal, new_dom

    return h_mal, h_dom
